# baseline (speedup 1.0000x reference)
import functools

import jax
import jax.numpy as jnp
from jax import lax
from jax.experimental import pallas as pl
from jax.experimental.pallas import tpu as pltpu, tpu_sc as plsc

_NW = 32
_L = 16
_U = 8
_LAG = 4


def _embedding_sc(B, b_per_w, D):
    n_iters = b_per_w // _U
    mesh = plsc.VectorSubcoreMesh(core_axis_name="c", subcore_axis_name="s")

    @functools.partial(
        pl.kernel,
        mesh=mesh,
        out_type=jax.ShapeDtypeStruct((B, D), jnp.float32),
        scratch_types=[
            pltpu.VMEM((b_per_w,), jnp.int32),
            pltpu.SMEM((b_per_w,), jnp.int32),
            pltpu.SemaphoreType.DMA,
        ],
    )
    def k(ids_hbm, table_hbm, out_hbm, ids_v, ids_s, sem):
        nc = lax.axis_size("c")
        wid = lax.axis_index("s") * nc + lax.axis_index("c")
        base = wid * b_per_w
        pltpu.sync_copy(ids_hbm.at[wid], ids_v)

        def unpack_body(c, _):
            vec = ids_v[pl.ds(c * _L, _L)]
            for u in range(_L):
                ids_s[c * _L + u] = vec[u]
            return ()

        lax.fori_loop(0, b_per_w // _L, unpack_body, ())

        def row_copy(i):
            v = ids_s[i]
            return pltpu.make_async_copy(
                table_hbm.at[v],
                out_hbm.at[base + i],
                sem,
            )

        def issue_body(it, _):
            for u in range(_U):
                row_copy(it * _U + u).start()

            @pl.when(it >= _LAG)
            def _():
                for u in range(_U):
                    row_copy((it - _LAG) * _U + u).wait()

            return ()

        lax.fori_loop(0, n_iters, issue_body, ())
        for t in range(_LAG):
            for u in range(_U):
                row_copy((n_iters - _LAG + t) * _U + u).wait()

    return k


def kernel(input_ids, table):
    B = input_ids.shape[0]
    D = table.shape[1]
    b_per_w = B // _NW
    ids2 = input_ids.astype(jnp.int32).reshape(_NW, b_per_w)
    out = _embedding_sc(B, b_per_w, D)(ids2, table)
    return out.reshape(B, 1, D)


# speedup vs baseline: 1.3414x; 1.3414x over previous
import functools

import jax
import jax.numpy as jnp
from jax import lax
from jax.experimental import pallas as pl
from jax.experimental.pallas import tpu as pltpu, tpu_sc as plsc

_NW = 32
_L = 16
_U = 8
_LAG = 4


def _embedding_sc(B, b_per_w, D):
    n_iters = b_per_w // _U
    pack = 128 // D
    mesh = plsc.VectorSubcoreMesh(core_axis_name="c", subcore_axis_name="s")

    @functools.partial(
        pl.kernel,
        mesh=mesh,
        out_type=jax.ShapeDtypeStruct((_NW, b_per_w // (128 // D), 128), jnp.float32),
        scratch_types=[
            pltpu.VMEM((b_per_w,), jnp.int32),
            pltpu.SMEM((b_per_w,), jnp.int32),
            pltpu.SemaphoreType.DMA,
        ],
    )
    def k(ids_hbm, table_hbm, out_hbm, ids_v, ids_s, sem):
        nc = lax.axis_size("c")
        wid = lax.axis_index("s") * nc + lax.axis_index("c")
        pltpu.sync_copy(ids_hbm.at[wid], ids_v)

        def unpack_body(c, _):
            vec = ids_v[pl.ds(c * _L, _L)]
            for u in range(_L):
                ids_s[c * _L + u] = vec[u]
            return ()

        lax.fori_loop(0, b_per_w // _L, unpack_body, ())

        def row_copy(i):
            v = ids_s[i]
            return pltpu.make_async_copy(
                table_hbm.at[v],
                out_hbm.at[wid, i // pack, pl.ds((i % pack) * D, D)],
                sem,
            )

        def issue_body(it, _):
            for u in range(_U):
                row_copy(it * _U + u).start()

            @pl.when(it >= _LAG)
            def _():
                for u in range(_U):
                    row_copy((it - _LAG) * _U + u).wait()

            return ()

        lax.fori_loop(0, n_iters, issue_body, ())
        for t in range(_LAG):
            for u in range(_U):
                row_copy((n_iters - _LAG + t) * _U + u).wait()

    return k


def kernel(input_ids, table):
    B = input_ids.shape[0]
    D = table.shape[1]
    b_per_w = B // _NW
    ids2 = input_ids.astype(jnp.int32).reshape(_NW, b_per_w)
    out = _embedding_sc(B, b_per_w, D)(ids2, table)
    return out.reshape(B, 1, D)


# speedup vs baseline: 1.7318x; 1.2911x over previous
import functools

import jax
import jax.numpy as jnp
from jax import lax
from jax.experimental import pallas as pl
from jax.experimental.pallas import tpu as pltpu, tpu_sc as plsc

_NW = 32
_L = 16
_U = 8
_LAG = 4


def _embedding_sc(B, b_per_w, D):
    n_iters = b_per_w // _U
    pack = 128 // D
    mesh = plsc.VectorSubcoreMesh(core_axis_name="c", subcore_axis_name="s")

    @functools.partial(
        pl.kernel,
        mesh=mesh,
        out_type=jax.ShapeDtypeStruct((_NW, b_per_w // (128 // D), 128), jnp.float32),
        scratch_types=[
            pltpu.VMEM((b_per_w,), jnp.int32),
            pltpu.SMEM((b_per_w,), jnp.int32),
            pltpu.VMEM((b_per_w // (128 // D), 128), jnp.float32),
            pltpu.SemaphoreType.DMA,
        ],
    )
    def k(ids_hbm, table_hbm, out_hbm, ids_v, ids_s, rows_v, sem):
        nc = lax.axis_size("c")
        wid = lax.axis_index("s") * nc + lax.axis_index("c")
        pltpu.sync_copy(ids_hbm.at[wid], ids_v)

        def unpack_body(c, _):
            vec = ids_v[pl.ds(c * _L, _L)]
            for u in range(_L):
                ids_s[c * _L + u] = vec[u]
            return ()

        lax.fori_loop(0, b_per_w // _L, unpack_body, ())

        def row_copy(i):
            v = ids_s[i]
            return pltpu.make_async_copy(
                table_hbm.at[v],
                rows_v.at[i // pack, pl.ds((i % pack) * D, D)],
                sem,
            )

        def issue_body(it, _):
            for u in range(_U):
                row_copy(it * _U + u).start()

            @pl.when(it >= _LAG)
            def _():
                for u in range(_U):
                    row_copy((it - _LAG) * _U + u).wait()

            return ()

        lax.fori_loop(0, n_iters, issue_body, ())
        for t in range(_LAG):
            for u in range(_U):
                row_copy((n_iters - _LAG + t) * _U + u).wait()
        pltpu.sync_copy(rows_v, out_hbm.at[wid])

    return k


def kernel(input_ids, table):
    B = input_ids.shape[0]
    D = table.shape[1]
    b_per_w = B // _NW
    ids2 = input_ids.astype(jnp.int32).reshape(_NW, b_per_w)
    out = _embedding_sc(B, b_per_w, D)(ids2, table)
    return out.reshape(B, 1, D)


# speedup vs baseline: 1.7438x; 1.0070x over previous
import functools

import jax
import jax.numpy as jnp
from jax import lax
from jax.experimental import pallas as pl
from jax.experimental.pallas import tpu as pltpu, tpu_sc as plsc

_NW = 32
_L = 16
_U = 8
_LAG = 8


def _embedding_sc(B, b_per_w, D):
    n_iters = b_per_w // _U
    pack = 128 // D
    mesh = plsc.VectorSubcoreMesh(core_axis_name="c", subcore_axis_name="s")

    @functools.partial(
        pl.kernel,
        mesh=mesh,
        out_type=jax.ShapeDtypeStruct((_NW, b_per_w // (128 // D), 128), jnp.float32),
        scratch_types=[
            pltpu.VMEM((b_per_w,), jnp.int32),
            pltpu.SMEM((b_per_w,), jnp.int32),
            pltpu.VMEM((b_per_w // (128 // D), 128), jnp.float32),
            pltpu.SemaphoreType.DMA,
        ],
    )
    def k(ids_hbm, table_hbm, out_hbm, ids_v, ids_s, rows_v, sem):
        nc = lax.axis_size("c")
        wid = lax.axis_index("s") * nc + lax.axis_index("c")
        pltpu.sync_copy(ids_hbm.at[wid], ids_v)

        def unpack_body(c, _):
            vec = ids_v[pl.ds(c * _L, _L)]
            for u in range(_L):
                ids_s[c * _L + u] = vec[u]
            return ()

        lax.fori_loop(0, b_per_w // _L, unpack_body, ())

        def row_copy(i):
            v = ids_s[i]
            return pltpu.make_async_copy(
                table_hbm.at[v],
                rows_v.at[i // pack, pl.ds((i % pack) * D, D)],
                sem,
            )

        def issue_body(it, _):
            for u in range(_U):
                row_copy(it * _U + u).start()

            @pl.when(it >= _LAG)
            def _():
                for u in range(_U):
                    row_copy((it - _LAG) * _U + u).wait()

            return ()

        lax.fori_loop(0, n_iters, issue_body, ())
        for t in range(_LAG):
            for u in range(_U):
                row_copy((n_iters - _LAG + t) * _U + u).wait()
        pltpu.sync_copy(rows_v, out_hbm.at[wid])

    return k


def kernel(input_ids, table):
    B = input_ids.shape[0]
    D = table.shape[1]
    b_per_w = B // _NW
    ids2 = input_ids.astype(jnp.int32).reshape(_NW, b_per_w)
    out = _embedding_sc(B, b_per_w, D)(ids2, table)
    return out.reshape(B, 1, D)
